# baseline (speedup 1.0000x reference)
import numpy as np

import jax
import jax.numpy as jnp
from jax.experimental import pallas as pl
from jax.experimental.pallas import tpu as pltpu
from jax.experimental.pallas import tpu_sc as plsc
from jax.sharding import Mesh, PartitionSpec as P

try:
    from jax import shard_map as _shard_map_fn

    def _shard_map(f, mesh, in_specs, out_specs):
        return _shard_map_fn(f, mesh=mesh, in_specs=in_specs,
                             out_specs=out_specs, check_vma=False)
except ImportError:
    from jax.experimental.shard_map import shard_map as _shard_map_fn

    def _shard_map(f, mesh, in_specs, out_specs):
        return _shard_map_fn(f, mesh=mesh, in_specs=in_specs,
                             out_specs=out_specs, check_vma=False)

_NUM_EMB = 8192
_HALF = _NUM_EMB // 2
_DIM = 32
_ROW_BLOCK = 1024
_GATHER_WINDOW = 128


def _vq_body(x_ref, emb_ref, idx_ref, loss_ref, em2_ref, esq_ref):
    i = pl.program_id(0)

    @pl.when(i == 0)
    def _precompute():
        emb = emb_ref[...]
        em2_ref[...] = -2.0 * emb
        esq_ref[...] = jnp.sum(emb * emb, axis=1)[None, :]
        loss_ref[...] = jnp.zeros((1, 1), jnp.float32)

    x = x_ref[...]
    isq = jnp.sum(x * x, axis=1, keepdims=True)
    mm2 = jax.lax.dot_general(
        x, em2_ref[...], dimension_numbers=(((1,), (1,)), ((), ())),
        preferred_element_type=jnp.float32)
    d = (isq + mm2) + esq_ref[...]

    d_lo = d[:, :_HALF]
    d_hi = d[:, _HALF:]
    v_lo = jnp.min(d_lo, axis=1, keepdims=True)
    v_hi = jnp.min(d_hi, axis=1, keepdims=True)
    am_lo = jnp.argmin(d_lo, axis=1).astype(jnp.int32)
    am_hi = jnp.argmin(d_hi, axis=1).astype(jnp.int32)

    key_lo = v_lo.astype(jnp.bfloat16).astype(jnp.float32)
    key_hi = jax.lax.bitcast_convert_type(
        jax.lax.bitcast_convert_type(v_hi, jnp.uint32)
        & jnp.uint32(0xFFFF8000), jnp.float32)
    pick_lo = key_lo <= key_hi
    am = jnp.where(pick_lo[:, 0], am_lo, am_hi + _HALF)
    idx_ref[...] = am[:, None]

    picked_v = jnp.where(pick_lo, v_lo, v_hi)
    loss_ref[...] += jnp.sum(picked_v).reshape(1, 1)


def _tc_part(flat_x, embeddings):
    rows = flat_x.shape[0]
    grid = rows // _ROW_BLOCK
    return pl.pallas_call(
        _vq_body,
        grid=(grid,),
        in_specs=[
            pl.BlockSpec((_ROW_BLOCK, _DIM), lambda i: (i, 0)),
            pl.BlockSpec((_NUM_EMB, _DIM), lambda i: (0, 0)),
        ],
        out_specs=[
            pl.BlockSpec((_ROW_BLOCK, 1), lambda i: (i, 0)),
            pl.BlockSpec((1, 1), lambda i: (0, 0)),
        ],
        out_shape=[
            jax.ShapeDtypeStruct((rows, 1), jnp.int32),
            jax.ShapeDtypeStruct((1, 1), jnp.float32),
        ],
        scratch_shapes=[
            pltpu.VMEM((_NUM_EMB, _DIM), jnp.float32),
            pltpu.VMEM((1, _NUM_EMB), jnp.float32),
        ],
    )(flat_x, embeddings)


def _sc_gather(emb_padded, idx_row):
    rows = idx_row.shape[1]
    width = emb_padded.shape[1]
    mesh = plsc.VectorSubcoreMesh(core_axis_name="core",
                                  subcore_axis_name="subcore")

    @pl.kernel(out_type=jax.ShapeDtypeStruct((rows, width), emb_padded.dtype),
               mesh=mesh)
    def gather_kernel(x_hbm, i_hbm, o_hbm):
        def body(i_vmem, o_vmem):
            pltpu.sync_copy(x_hbm.at[i_vmem.at[0]], o_vmem)

        pltpu.emit_pipeline(
            body,
            grid=(rows // _GATHER_WINDOW,),
            in_specs=[pl.BlockSpec((1, _GATHER_WINDOW),
                                   index_map=lambda i: (0, i))],
            out_specs=[pl.BlockSpec((_GATHER_WINDOW, width),
                                    index_map=lambda i: (i, 0))],
            core_axis_name="subcore",
            dimension_semantics=(pltpu.PARALLEL,),
        )(i_hbm, o_hbm)

    return gather_kernel(emb_padded, idx_row)


def _shard_fn(xs, emb):
    rows_local = xs.shape[0]
    idx, loss_part = _tc_part(xs, emb)
    emb_padded = jnp.pad(emb, ((0, 0), (0, 128 - _DIM)))
    gathered = _sc_gather(emb_padded, idx.reshape(1, rows_local))
    return idx, loss_part, gathered[:, :_DIM]


def kernel(inputs, embeddings):
    in_shape = inputs.shape
    rows = in_shape[0] * in_shape[1]
    flat_x = inputs.reshape(rows, _DIM)
    idx, loss_parts, quantized = _shard_fn(flat_x, embeddings)
    loss = jnp.sum(loss_parts) / jnp.float32(rows * _DIM)
    encoding_indices = idx.reshape(in_shape[:-1])
    return (quantized.reshape(in_shape), loss, encoding_indices)


# speedup vs baseline: 1.0050x; 1.0050x over previous
import jax
import jax.numpy as jnp
from jax.experimental import pallas as pl
from jax.experimental.pallas import tpu as pltpu
from jax.experimental.pallas import tpu_sc as plsc

_NUM_EMB = 8192
_HALF = _NUM_EMB // 2
_DIM = 32
_ROW_BLOCK = 512
_GATHER_WINDOW = 128


def _vq_body(x_ref, emb_ref, idx_ref, loss_ref, em2_ref, esq_ref):
    i = pl.program_id(0)

    @pl.when(i == 0)
    def _precompute():
        emb = emb_ref[...]
        em2_ref[...] = -2.0 * emb
        esq_ref[...] = jnp.sum(emb * emb, axis=1)[None, :]
        loss_ref[...] = jnp.zeros((1, 1), jnp.float32)

    x = x_ref[...]
    isq = jnp.sum(x * x, axis=1, keepdims=True)
    mm2 = jax.lax.dot_general(
        x, em2_ref[...], dimension_numbers=(((1,), (1,)), ((), ())),
        preferred_element_type=jnp.float32)
    d = (isq + mm2) + esq_ref[...]

    d_lo = d[:, :_HALF]
    d_hi = d[:, _HALF:]
    v_lo = jnp.min(d_lo, axis=1, keepdims=True)
    v_hi = jnp.min(d_hi, axis=1, keepdims=True)
    am_lo = jnp.argmin(d_lo, axis=1).astype(jnp.int32)
    am_hi = jnp.argmin(d_hi, axis=1).astype(jnp.int32)

    key_lo = v_lo.astype(jnp.bfloat16).astype(jnp.float32)
    key_hi = jax.lax.bitcast_convert_type(
        jax.lax.bitcast_convert_type(v_hi, jnp.uint32)
        & jnp.uint32(0xFFFF8000), jnp.float32)
    pick_lo = key_lo <= key_hi
    am = jnp.where(pick_lo[:, 0], am_lo, am_hi + _HALF)
    idx_ref[...] = am[:, None]

    picked_v = jnp.where(pick_lo, v_lo, v_hi)
    loss_ref[...] += jnp.sum(picked_v).reshape(1, 1)


def _tc_part(flat_x, embeddings):
    rows = flat_x.shape[0]
    grid = rows // _ROW_BLOCK
    return pl.pallas_call(
        _vq_body,
        grid=(grid,),
        in_specs=[
            pl.BlockSpec((_ROW_BLOCK, _DIM), lambda i: (i, 0)),
            pl.BlockSpec((_NUM_EMB, _DIM), lambda i: (0, 0)),
        ],
        out_specs=[
            pl.BlockSpec((_ROW_BLOCK, 1), lambda i: (i, 0)),
            pl.BlockSpec((1, 1), lambda i: (0, 0)),
        ],
        out_shape=[
            jax.ShapeDtypeStruct((rows, 1), jnp.int32),
            jax.ShapeDtypeStruct((1, 1), jnp.float32),
        ],
        scratch_shapes=[
            pltpu.VMEM((_NUM_EMB, _DIM), jnp.float32),
            pltpu.VMEM((1, _NUM_EMB), jnp.float32),
        ],
    )(flat_x, embeddings)


def _sc_gather(emb_padded, idx_row):
    rows = idx_row.shape[1]
    width = emb_padded.shape[1]
    mesh = plsc.VectorSubcoreMesh(core_axis_name="core",
                                  subcore_axis_name="subcore")

    @pl.kernel(out_type=jax.ShapeDtypeStruct((rows, width), emb_padded.dtype),
               mesh=mesh)
    def gather_kernel(x_hbm, i_hbm, o_hbm):
        def body(i_vmem, o_vmem):
            pltpu.sync_copy(x_hbm.at[i_vmem.at[0]], o_vmem)

        pltpu.emit_pipeline(
            body,
            grid=(rows // _GATHER_WINDOW,),
            in_specs=[pl.BlockSpec((1, _GATHER_WINDOW),
                                   index_map=lambda i: (0, i))],
            out_specs=[pl.BlockSpec((_GATHER_WINDOW, width),
                                    index_map=lambda i: (i, 0))],
            core_axis_name="subcore",
            dimension_semantics=(pltpu.PARALLEL,),
        )(i_hbm, o_hbm)

    return gather_kernel(emb_padded, idx_row)


def _shard_fn(xs, emb):
    rows_local = xs.shape[0]
    idx, loss_part = _tc_part(xs, emb)
    emb_padded = jnp.pad(emb, ((0, 0), (0, 128 - _DIM)))
    gathered = _sc_gather(emb_padded, idx.reshape(1, rows_local))
    return idx, loss_part, gathered[:, :_DIM]


def kernel(inputs, embeddings):
    in_shape = inputs.shape
    rows = in_shape[0] * in_shape[1]
    flat_x = inputs.reshape(rows, _DIM)
    idx, loss_parts, quantized = _shard_fn(flat_x, embeddings)
    loss = jnp.sum(loss_parts) / jnp.float32(rows * _DIM)
    encoding_indices = idx.reshape(in_shape[:-1])
    return (quantized.reshape(in_shape), loss, encoding_indices)
